# Initial kernel scaffold; baseline (speedup 1.0000x reference)
import jax
import jax.numpy as jnp
from jax.experimental import pallas as pl


def kernel(features, edge_index, W_l1, b1, W_r1, W_l2, b2, W_r2):
    raise NotImplementedError("write your pallas kernel here")


# speedup vs baseline: 5.7479x; 5.7479x over previous
import functools

import jax
import jax.numpy as jnp
from jax import lax
from jax.experimental import pallas as pl
from jax.experimental.pallas import tpu as pltpu
from jax.experimental.pallas import tpu_sc as plsc

N = 10000
E = 320000
D_IN = 128
D_H = 128
N_CLS = 64

NC = 2
NS = 16
NW = NC * NS
CH = 128
K = 80
EPAD = NW * K * CH
NACC = 10240
RPT = NACC // NS


def _zero_rows(rows_v, D):
  def zrow(r, _):
    for c in range(D // 16):
      rows_v[r, pl.ds(c * 16, 16)] = jnp.zeros((16,), jnp.float32)
    return 0
  lax.fori_loop(0, CH, zrow, 0)


@functools.cache
def _sc_segsum(D: int):
  mesh = plsc.VectorSubcoreMesh(core_axis_name="c", subcore_axis_name="s")

  def body(table, src3, dst3, out, src_v, dst_v, rows_v, acc_sh, sem):
    cid = lax.axis_index("c")
    sid = lax.axis_index("s")
    wid = sid * NC + cid

    _zero_rows(rows_v, D)
    for i in range(RPT // CH):
      pltpu.sync_copy(rows_v, acc_sh.at[pl.ds(sid * RPT + i * CH, CH)])

    plsc.subcore_barrier()

    def edge_chunk(j, _):
      pltpu.sync_copy(src3.at[wid, j], src_v)
      pltpu.sync_copy(dst3.at[wid, j], dst_v)
      pltpu.async_copy(table.at[src_v], rows_v, sem).wait()
      pltpu.sync_copy(rows_v, acc_sh.at[dst_v], add=True)
      return 0
    lax.fori_loop(0, K, edge_chunk, 0)

    plsc.subcore_barrier()

    r0 = sid * RPT
    pltpu.sync_copy(acc_sh.at[pl.ds(r0, RPT)], out.at[cid, pl.ds(r0, RPT)])

  return pl.kernel(
      body,
      out_type=jax.ShapeDtypeStruct((NC, NACC, D), jnp.float32),
      mesh=mesh,
      scratch_types=[
          pltpu.VMEM((CH,), jnp.int32),
          pltpu.VMEM((CH,), jnp.int32),
          pltpu.VMEM((CH, D), jnp.float32),
          pltpu.VMEM_SHARED((NACC, D), jnp.float32),
          pltpu.SemaphoreType.DMA,
      ],
  )


@functools.cache
def _sc_degree():
  mesh = plsc.VectorSubcoreMesh(core_axis_name="c", subcore_axis_name="s")

  def body(dst3, out, dst_v, ones_v, acc_sh):
    cid = lax.axis_index("c")
    sid = lax.axis_index("s")
    wid = sid * NC + cid

    _zero_rows(ones_v, 128)
    for i in range(RPT // CH):
      pltpu.sync_copy(ones_v, acc_sh.at[pl.ds(sid * RPT + i * CH, CH)])
    def orow(r, _):
      for c in range(128 // 16):
        ones_v[r, pl.ds(c * 16, 16)] = jnp.ones((16,), jnp.float32)
      return 0
    lax.fori_loop(0, CH, orow, 0)

    plsc.subcore_barrier()

    def edge_chunk(j, _):
      pltpu.sync_copy(dst3.at[wid, j], dst_v)
      pltpu.sync_copy(ones_v, acc_sh.at[dst_v], add=True)
      return 0
    lax.fori_loop(0, K, edge_chunk, 0)

    plsc.subcore_barrier()

    r0 = sid * RPT
    pltpu.sync_copy(acc_sh.at[pl.ds(r0, RPT)], out.at[cid, pl.ds(r0, RPT)])

  return pl.kernel(
      body,
      out_type=jax.ShapeDtypeStruct((NC, NACC, 128), jnp.float32),
      mesh=mesh,
      scratch_types=[
          pltpu.VMEM((CH,), jnp.int32),
          pltpu.VMEM((CH, 128), jnp.float32),
          pltpu.VMEM_SHARED((NACC, 128), jnp.float32),
      ],
  )


_BR = 1000
_HIGH = jax.lax.Precision.HIGHEST


def _tc_proj1(x_ref, wl_ref, wr_ref, b_ref, p_ref, r_ref):
  x = x_ref[...]
  p_ref[...] = jnp.dot(x, wl_ref[...], precision=_HIGH,
                       preferred_element_type=jnp.float32)
  r_ref[...] = jnp.dot(x, wr_ref[...], precision=_HIGH,
                       preferred_element_type=jnp.float32) + b_ref[...]


def _tc_mid(s_ref, deg_ref, r1_ref, wr2_ref, b2_ref, h_ref, r2_ref):
  s = s_ref[0] + s_ref[1]
  deg = deg_ref[0, :, 0] + deg_ref[1, :, 0]
  degc = jnp.maximum(deg, 1.0)[:, None]
  h = jnp.maximum(s / degc + r1_ref[...], 0.0)
  h_ref[...] = h
  r2_ref[...] = jnp.dot(h, wr2_ref[...], precision=_HIGH,
                        preferred_element_type=jnp.float32) + b2_ref[...]


def _tc_out(s_ref, deg_ref, r2_ref, wl2_ref, o_ref):
  s = s_ref[0] + s_ref[1]
  deg = deg_ref[0, :, 0] + deg_ref[1, :, 0]
  degc = jnp.maximum(deg, 1.0)[:, None]
  z = jnp.dot(s / degc, wl2_ref[...], precision=_HIGH,
              preferred_element_type=jnp.float32) + r2_ref[...]
  m = jnp.max(z, axis=-1, keepdims=True)
  e = z - m
  lse = jnp.log(jnp.sum(jnp.exp(e), axis=-1, keepdims=True))
  o_ref[...] = e - lse


def kernel(features, edge_index, W_l1, b1, W_r1, W_l2, b2, W_r2):
  src = edge_index[0].astype(jnp.int32)
  dst = edge_index[1].astype(jnp.int32)
  pad = EPAD - E
  ar = jnp.arange(pad, dtype=jnp.int32)
  srcp = jnp.concatenate([src, ar % N]).reshape(NW, K, CH)
  dstp = jnp.concatenate([dst, N + ar % (NACC - N)]).reshape(NW, K, CH)

  nb = N // _BR
  full2 = pl.BlockSpec((_BR, D_IN), lambda i: (i, 0))
  wspec = pl.BlockSpec((D_IN, D_H), lambda i: (0, 0))

  p1, r1 = pl.pallas_call(
      _tc_proj1,
      grid=(nb,),
      in_specs=[full2, wspec, wspec, pl.BlockSpec((1, D_H), lambda i: (0, 0))],
      out_specs=[pl.BlockSpec((_BR, D_H), lambda i: (i, 0))] * 2,
      out_shape=[jax.ShapeDtypeStruct((N, D_H), jnp.float32)] * 2,
  )(features, W_l1, W_r1, b1.reshape(1, D_H))

  deg = _sc_degree()(dstp)
  p1, srcp, dstp, deg = lax.optimization_barrier((p1, srcp, dstp, deg))
  s1 = _sc_segsum(D_H)(p1, srcp, dstp)

  h, r2 = pl.pallas_call(
      _tc_mid,
      grid=(nb,),
      in_specs=[
          pl.BlockSpec((NC, _BR, D_H), lambda i: (0, i, 0)),
          pl.BlockSpec((NC, _BR, 128), lambda i: (0, i, 0)),
          pl.BlockSpec((_BR, D_H), lambda i: (i, 0)),
          pl.BlockSpec((D_H, N_CLS), lambda i: (0, 0)),
          pl.BlockSpec((1, N_CLS), lambda i: (0, 0)),
      ],
      out_specs=[pl.BlockSpec((_BR, D_H), lambda i: (i, 0)),
                 pl.BlockSpec((_BR, N_CLS), lambda i: (i, 0))],
      out_shape=[jax.ShapeDtypeStruct((N, D_H), jnp.float32),
                 jax.ShapeDtypeStruct((N, N_CLS), jnp.float32)],
  )(s1, deg, r1, W_r2, b2.reshape(1, N_CLS))

  s2 = _sc_segsum(D_H)(h, srcp, dstp)

  out = pl.pallas_call(
      _tc_out,
      grid=(nb,),
      in_specs=[
          pl.BlockSpec((NC, _BR, D_H), lambda i: (0, i, 0)),
          pl.BlockSpec((NC, _BR, 128), lambda i: (0, i, 0)),
          pl.BlockSpec((_BR, N_CLS), lambda i: (i, 0)),
          pl.BlockSpec((D_H, N_CLS), lambda i: (0, 0)),
      ],
      out_specs=pl.BlockSpec((_BR, N_CLS), lambda i: (i, 0)),
      out_shape=jax.ShapeDtypeStruct((N, N_CLS), jnp.float32),
  )(s2, deg, r2, W_l2)
  return out


# speedup vs baseline: 8.7833x; 1.5281x over previous
import functools

import jax
import jax.numpy as jnp
from jax import lax
from jax.experimental import pallas as pl
from jax.experimental.pallas import tpu as pltpu
from jax.experimental.pallas import tpu_sc as plsc

N = 10000
E = 320000
D_IN = 128
D_H = 128
N_CLS = 64

NC = 2
NS = 16
NW = NC * NS
CH = 128
K = 80
EPAD = NW * K * CH
NACC = 10240
RPT = NACC // NS


def _zero_rows(rows_v, D):
  def zrow(r, _):
    for c in range(D // 16):
      rows_v[r, pl.ds(c * 16, 16)] = jnp.zeros((16,), jnp.float32)
    return 0
  lax.fori_loop(0, CH, zrow, 0)


@functools.cache
def _sc_segsum(D: int):
  mesh = plsc.VectorSubcoreMesh(core_axis_name="c", subcore_axis_name="s")

  def body(table, src3, dst3, out, src_v, dst_v, rows_v, acc_sh,
           sem0, sem1):
    cid = lax.axis_index("c")
    sid = lax.axis_index("s")
    wid = sid * NC + cid
    sems = (sem0, sem1)

    _zero_rows(rows_v.at[0], D)
    for i in range(RPT // CH):
      pltpu.sync_copy(rows_v.at[0], acc_sh.at[pl.ds(sid * RPT + i * CH, CH)])

    plsc.subcore_barrier()

    for b in range(2):
      pltpu.sync_copy(src3.at[wid, b], src_v.at[b])
      pltpu.sync_copy(dst3.at[wid, b], dst_v.at[b])
      pltpu.async_copy(table.at[src_v.at[b]], rows_v.at[b], sems[b])

    def pair(g, _):
      for b in range(2):
        j = 2 * g + b
        pltpu.make_async_copy(table.at[src_v.at[b]], rows_v.at[b],
                              sems[b]).wait()
        pltpu.sync_copy(rows_v.at[b], acc_sh.at[dst_v.at[b]], add=True)
        @pl.when(j + 2 < K)
        def _():
          pltpu.sync_copy(src3.at[wid, j + 2], src_v.at[b])
          pltpu.sync_copy(dst3.at[wid, j + 2], dst_v.at[b])
          pltpu.async_copy(table.at[src_v.at[b]], rows_v.at[b], sems[b])
      return 0
    lax.fori_loop(0, K // 2, pair, 0)

    plsc.subcore_barrier()

    r0 = sid * RPT
    pltpu.sync_copy(acc_sh.at[pl.ds(r0, RPT)], out.at[cid, pl.ds(r0, RPT)])

  return pl.kernel(
      body,
      out_type=jax.ShapeDtypeStruct((NC, NACC, D), jnp.float32),
      mesh=mesh,
      scratch_types=[
          pltpu.VMEM((2, CH), jnp.int32),
          pltpu.VMEM((2, CH), jnp.int32),
          pltpu.VMEM((2, CH, D), jnp.float32),
          pltpu.VMEM_SHARED((NACC, D), jnp.float32),
          pltpu.SemaphoreType.DMA,
          pltpu.SemaphoreType.DMA,
      ],
  )


@functools.cache
def _sc_degree():
  mesh = plsc.VectorSubcoreMesh(core_axis_name="c", subcore_axis_name="s")

  def body(dst3, out, dst_v, ones_v, acc_sh, sem0, sem1):
    cid = lax.axis_index("c")
    sid = lax.axis_index("s")
    wid = sid * NC + cid
    sems = (sem0, sem1)

    _zero_rows(ones_v, 128)
    for i in range(RPT // CH):
      pltpu.sync_copy(ones_v, acc_sh.at[pl.ds(sid * RPT + i * CH, CH)])
    def orow(r, _):
      for c in range(128 // 16):
        ones_v[r, pl.ds(c * 16, 16)] = jnp.ones((16,), jnp.float32)
      return 0
    lax.fori_loop(0, CH, orow, 0)

    plsc.subcore_barrier()

    for b in range(2):
      pltpu.async_copy(dst3.at[wid, b], dst_v.at[b], sems[b])

    def pair(g, _):
      for b in range(2):
        j = 2 * g + b
        pltpu.make_async_copy(dst3.at[wid, j], dst_v.at[b], sems[b]).wait()
        pltpu.sync_copy(ones_v, acc_sh.at[dst_v.at[b]], add=True)
        @pl.when(j + 2 < K)
        def _():
          pltpu.async_copy(dst3.at[wid, j + 2], dst_v.at[b], sems[b])
      return 0
    lax.fori_loop(0, K // 2, pair, 0)

    plsc.subcore_barrier()

    r0 = sid * RPT
    pltpu.sync_copy(acc_sh.at[pl.ds(r0, RPT)], out.at[cid, pl.ds(r0, RPT)])

  return pl.kernel(
      body,
      out_type=jax.ShapeDtypeStruct((NC, NACC, 128), jnp.float32),
      mesh=mesh,
      scratch_types=[
          pltpu.VMEM((2, CH), jnp.int32),
          pltpu.VMEM((CH, 128), jnp.float32),
          pltpu.VMEM_SHARED((NACC, 128), jnp.float32),
          pltpu.SemaphoreType.DMA,
          pltpu.SemaphoreType.DMA,
      ],
  )


_BR = 1000
_HIGH = jax.lax.Precision.HIGHEST


def _tc_proj1(x_ref, wl_ref, wr_ref, b_ref, p_ref, r_ref):
  x = x_ref[...]
  p_ref[...] = jnp.dot(x, wl_ref[...], precision=_HIGH,
                       preferred_element_type=jnp.float32)
  r_ref[...] = jnp.dot(x, wr_ref[...], precision=_HIGH,
                       preferred_element_type=jnp.float32) + b_ref[...]


def _tc_mid(s_ref, deg_ref, r1_ref, wr2_ref, b2_ref, h_ref, r2_ref):
  s = s_ref[0] + s_ref[1]
  deg = deg_ref[0, :, 0] + deg_ref[1, :, 0]
  degc = jnp.maximum(deg, 1.0)[:, None]
  h = jnp.maximum(s / degc + r1_ref[...], 0.0)
  h_ref[...] = h
  r2_ref[...] = jnp.dot(h, wr2_ref[...], precision=_HIGH,
                        preferred_element_type=jnp.float32) + b2_ref[...]


def _tc_out(s_ref, deg_ref, r2_ref, wl2_ref, o_ref):
  s = s_ref[0] + s_ref[1]
  deg = deg_ref[0, :, 0] + deg_ref[1, :, 0]
  degc = jnp.maximum(deg, 1.0)[:, None]
  z = jnp.dot(s / degc, wl2_ref[...], precision=_HIGH,
              preferred_element_type=jnp.float32) + r2_ref[...]
  m = jnp.max(z, axis=-1, keepdims=True)
  e = z - m
  lse = jnp.log(jnp.sum(jnp.exp(e), axis=-1, keepdims=True))
  o_ref[...] = e - lse


def kernel(features, edge_index, W_l1, b1, W_r1, W_l2, b2, W_r2):
  src = edge_index[0].astype(jnp.int32)
  dst = edge_index[1].astype(jnp.int32)
  pad = EPAD - E
  ar = jnp.arange(pad, dtype=jnp.int32)
  srcp = jnp.concatenate([src, ar % N]).reshape(NW, K, CH)
  dstp = jnp.concatenate([dst, N + ar % (NACC - N)]).reshape(NW, K, CH)

  nb = N // _BR
  full2 = pl.BlockSpec((_BR, D_IN), lambda i: (i, 0))
  wspec = pl.BlockSpec((D_IN, D_H), lambda i: (0, 0))

  p1, r1 = pl.pallas_call(
      _tc_proj1,
      grid=(nb,),
      in_specs=[full2, wspec, wspec, pl.BlockSpec((1, D_H), lambda i: (0, 0))],
      out_specs=[pl.BlockSpec((_BR, D_H), lambda i: (i, 0))] * 2,
      out_shape=[jax.ShapeDtypeStruct((N, D_H), jnp.float32)] * 2,
  )(features, W_l1, W_r1, b1.reshape(1, D_H))

  deg = _sc_degree()(dstp)
  p1, srcp, dstp, deg = lax.optimization_barrier((p1, srcp, dstp, deg))
  s1 = _sc_segsum(D_H)(p1, srcp, dstp)

  h, r2 = pl.pallas_call(
      _tc_mid,
      grid=(nb,),
      in_specs=[
          pl.BlockSpec((NC, _BR, D_H), lambda i: (0, i, 0)),
          pl.BlockSpec((NC, _BR, 128), lambda i: (0, i, 0)),
          pl.BlockSpec((_BR, D_H), lambda i: (i, 0)),
          pl.BlockSpec((D_H, N_CLS), lambda i: (0, 0)),
          pl.BlockSpec((1, N_CLS), lambda i: (0, 0)),
      ],
      out_specs=[pl.BlockSpec((_BR, D_H), lambda i: (i, 0)),
                 pl.BlockSpec((_BR, N_CLS), lambda i: (i, 0))],
      out_shape=[jax.ShapeDtypeStruct((N, D_H), jnp.float32),
                 jax.ShapeDtypeStruct((N, N_CLS), jnp.float32)],
  )(s1, deg, r1, W_r2, b2.reshape(1, N_CLS))

  s2 = _sc_segsum(D_H)(h, srcp, dstp)

  out = pl.pallas_call(
      _tc_out,
      grid=(nb,),
      in_specs=[
          pl.BlockSpec((NC, _BR, D_H), lambda i: (0, i, 0)),
          pl.BlockSpec((NC, _BR, 128), lambda i: (0, i, 0)),
          pl.BlockSpec((_BR, N_CLS), lambda i: (i, 0)),
          pl.BlockSpec((D_H, N_CLS), lambda i: (0, 0)),
      ],
      out_specs=pl.BlockSpec((_BR, N_CLS), lambda i: (i, 0)),
      out_shape=jax.ShapeDtypeStruct((N, N_CLS), jnp.float32),
  )(s2, deg, r2, W_l2)
  return out


# speedup vs baseline: 10.1281x; 1.1531x over previous
import functools

import jax
import jax.numpy as jnp
from jax import lax
from jax.experimental import pallas as pl
from jax.experimental.pallas import tpu as pltpu
from jax.experimental.pallas import tpu_sc as plsc

N = 10000
E = 320000
D_IN = 128
D_H = 128
N_CLS = 64

NC = 2
NS = 16
NW = NC * NS
CH = 128
K = 81
EPAD = NW * K * CH
NACC = 10112
RPT = NACC // NS
NB = 3


def _zero_rows(rows_v, D):
  def zrow(r, _):
    for c in range(D // 16):
      rows_v[r, pl.ds(c * 16, 16)] = jnp.zeros((16,), jnp.float32)
    return 0
  lax.fori_loop(0, CH, zrow, 0)


def _zero_acc_slice(rows_v, acc_sh, sid):
  for i in range(RPT // CH):
    pltpu.sync_copy(rows_v, acc_sh.at[pl.ds(sid * RPT + i * CH, CH)])
  rem = RPT % CH
  if rem:
    pltpu.sync_copy(rows_v.at[pl.ds(0, rem)],
                    acc_sh.at[pl.ds(sid * RPT + (RPT // CH) * CH, rem)])


@functools.cache
def _sc_segsum(D: int):
  mesh = plsc.VectorSubcoreMesh(core_axis_name="c", subcore_axis_name="s")

  def body(table, src3, dst3, out, src_v, dst_v, rows_v, acc_sh,
           g0, g1, g2, s0, s1, s2):
    cid = lax.axis_index("c")
    sid = lax.axis_index("s")
    wid = sid * NC + cid
    gsem = (g0, g1, g2)
    ssem = (s0, s1, s2)

    _zero_rows(rows_v.at[0], D)
    _zero_acc_slice(rows_v.at[0], acc_sh, sid)

    plsc.subcore_barrier()

    def stage_and_gather(j, b):
      pltpu.sync_copy(src3.at[wid, j], src_v.at[b])
      pltpu.sync_copy(dst3.at[wid, j], dst_v.at[b])
      pltpu.async_copy(table.at[src_v.at[b]], rows_v.at[b], gsem[b])

    for b in range(2):
      stage_and_gather(b, b)

    def ring(g, _):
      for b in range(NB):
        j = NB * g + b
        bp = (b + 2) % NB
        pltpu.make_async_copy(table.at[src_v.at[b]], rows_v.at[b],
                              gsem[b]).wait()
        pltpu.async_copy(rows_v.at[b], acc_sh.at[dst_v.at[b]], ssem[b],
                         add=True)
        @pl.when(j + 2 < K)
        def _():
          @pl.when(j >= 1)
          def _():
            pltpu.make_async_copy(rows_v.at[bp], acc_sh.at[dst_v.at[bp]],
                                  ssem[bp]).wait()
          stage_and_gather(j + 2, bp)
      return 0
    lax.fori_loop(0, K // NB, ring, 0)

    for b in range(NB):
      pltpu.make_async_copy(rows_v.at[b], acc_sh.at[dst_v.at[b]],
                            ssem[b]).wait()

    plsc.subcore_barrier()

    r0 = sid * RPT
    pltpu.sync_copy(acc_sh.at[pl.ds(r0, RPT)], out.at[cid, pl.ds(r0, RPT)])

  return pl.kernel(
      body,
      out_type=jax.ShapeDtypeStruct((NC, NACC, D), jnp.float32),
      mesh=mesh,
      scratch_types=[
          pltpu.VMEM((NB, CH), jnp.int32),
          pltpu.VMEM((NB, CH), jnp.int32),
          pltpu.VMEM((NB, CH, D), jnp.float32),
          pltpu.VMEM_SHARED((NACC, D), jnp.float32),
          pltpu.SemaphoreType.DMA, pltpu.SemaphoreType.DMA,
          pltpu.SemaphoreType.DMA, pltpu.SemaphoreType.DMA,
          pltpu.SemaphoreType.DMA, pltpu.SemaphoreType.DMA,
      ],
  )


@functools.cache
def _sc_degree():
  mesh = plsc.VectorSubcoreMesh(core_axis_name="c", subcore_axis_name="s")

  def body(dst3, out, dst_v, ones_v, acc_sh, d0, d1, d2, s0, s1, s2):
    cid = lax.axis_index("c")
    sid = lax.axis_index("s")
    wid = sid * NC + cid
    dsem = (d0, d1, d2)
    ssem = (s0, s1, s2)

    _zero_rows(ones_v, 128)
    _zero_acc_slice(ones_v, acc_sh, sid)
    def orow(r, _):
      for c in range(128 // 16):
        ones_v[r, pl.ds(c * 16, 16)] = jnp.ones((16,), jnp.float32)
      return 0
    lax.fori_loop(0, CH, orow, 0)

    plsc.subcore_barrier()

    for b in range(2):
      pltpu.async_copy(dst3.at[wid, b], dst_v.at[b], dsem[b])

    def ring(g, _):
      for b in range(NB):
        j = NB * g + b
        bp = (b + 2) % NB
        pltpu.make_async_copy(dst3.at[wid, j], dst_v.at[b], dsem[b]).wait()
        pltpu.async_copy(ones_v, acc_sh.at[dst_v.at[b]], ssem[b], add=True)
        @pl.when(j + 2 < K)
        def _():
          @pl.when(j >= 1)
          def _():
            pltpu.make_async_copy(ones_v, acc_sh.at[dst_v.at[bp]],
                                  ssem[bp]).wait()
          pltpu.async_copy(dst3.at[wid, j + 2], dst_v.at[bp], dsem[bp])
      return 0
    lax.fori_loop(0, K // NB, ring, 0)

    for b in range(NB):
      pltpu.make_async_copy(ones_v, acc_sh.at[dst_v.at[b]], ssem[b]).wait()

    plsc.subcore_barrier()

    r0 = sid * RPT
    pltpu.sync_copy(acc_sh.at[pl.ds(r0, RPT)], out.at[cid, pl.ds(r0, RPT)])

  return pl.kernel(
      body,
      out_type=jax.ShapeDtypeStruct((NC, NACC, 128), jnp.float32),
      mesh=mesh,
      scratch_types=[
          pltpu.VMEM((NB, CH), jnp.int32),
          pltpu.VMEM((CH, 128), jnp.float32),
          pltpu.VMEM_SHARED((NACC, 128), jnp.float32),
          pltpu.SemaphoreType.DMA, pltpu.SemaphoreType.DMA,
          pltpu.SemaphoreType.DMA, pltpu.SemaphoreType.DMA,
          pltpu.SemaphoreType.DMA, pltpu.SemaphoreType.DMA,
      ],
  )


_BR = 1000
_HIGH = jax.lax.Precision.HIGHEST


def _tc_proj1(x_ref, wl_ref, wr_ref, b_ref, p_ref, r_ref):
  x = x_ref[...]
  p_ref[...] = jnp.dot(x, wl_ref[...], precision=_HIGH,
                       preferred_element_type=jnp.float32)
  r_ref[...] = jnp.dot(x, wr_ref[...], precision=_HIGH,
                       preferred_element_type=jnp.float32) + b_ref[...]


def _tc_mid(s_ref, deg_ref, r1_ref, wr2_ref, b2_ref, h_ref, r2_ref):
  s = s_ref[0] + s_ref[1]
  deg = deg_ref[0, :, 0] + deg_ref[1, :, 0]
  degc = jnp.maximum(deg, 1.0)[:, None]
  h = jnp.maximum(s / degc + r1_ref[...], 0.0)
  h_ref[...] = h
  r2_ref[...] = jnp.dot(h, wr2_ref[...], precision=_HIGH,
                        preferred_element_type=jnp.float32) + b2_ref[...]


def _tc_out(s_ref, deg_ref, r2_ref, wl2_ref, o_ref):
  s = s_ref[0] + s_ref[1]
  deg = deg_ref[0, :, 0] + deg_ref[1, :, 0]
  degc = jnp.maximum(deg, 1.0)[:, None]
  z = jnp.dot(s / degc, wl2_ref[...], precision=_HIGH,
              preferred_element_type=jnp.float32) + r2_ref[...]
  m = jnp.max(z, axis=-1, keepdims=True)
  e = z - m
  lse = jnp.log(jnp.sum(jnp.exp(e), axis=-1, keepdims=True))
  o_ref[...] = e - lse


def kernel(features, edge_index, W_l1, b1, W_r1, W_l2, b2, W_r2):
  src = edge_index[0].astype(jnp.int32)
  dst = edge_index[1].astype(jnp.int32)
  pad = EPAD - E
  ar = jnp.arange(pad, dtype=jnp.int32)
  srcp = jnp.concatenate([src, ar % N]).reshape(NW, K, CH)
  dstp = jnp.concatenate([dst, N + ar % (NACC - N)]).reshape(NW, K, CH)

  nb = N // _BR
  full2 = pl.BlockSpec((_BR, D_IN), lambda i: (i, 0))
  wspec = pl.BlockSpec((D_IN, D_H), lambda i: (0, 0))

  p1, r1 = pl.pallas_call(
      _tc_proj1,
      grid=(nb,),
      in_specs=[full2, wspec, wspec, pl.BlockSpec((1, D_H), lambda i: (0, 0))],
      out_specs=[pl.BlockSpec((_BR, D_H), lambda i: (i, 0))] * 2,
      out_shape=[jax.ShapeDtypeStruct((N, D_H), jnp.float32)] * 2,
  )(features, W_l1, W_r1, b1.reshape(1, D_H))

  deg = _sc_degree()(dstp)
  p1, srcp, dstp, deg = lax.optimization_barrier((p1, srcp, dstp, deg))
  s1 = _sc_segsum(D_H)(p1, srcp, dstp)

  h, r2 = pl.pallas_call(
      _tc_mid,
      grid=(nb,),
      in_specs=[
          pl.BlockSpec((NC, _BR, D_H), lambda i: (0, i, 0)),
          pl.BlockSpec((NC, _BR, 128), lambda i: (0, i, 0)),
          pl.BlockSpec((_BR, D_H), lambda i: (i, 0)),
          pl.BlockSpec((D_H, N_CLS), lambda i: (0, 0)),
          pl.BlockSpec((1, N_CLS), lambda i: (0, 0)),
      ],
      out_specs=[pl.BlockSpec((_BR, D_H), lambda i: (i, 0)),
                 pl.BlockSpec((_BR, N_CLS), lambda i: (i, 0))],
      out_shape=[jax.ShapeDtypeStruct((N, D_H), jnp.float32),
                 jax.ShapeDtypeStruct((N, N_CLS), jnp.float32)],
  )(s1, deg, r1, W_r2, b2.reshape(1, N_CLS))

  s2 = _sc_segsum(D_H)(h, srcp, dstp)

  out = pl.pallas_call(
      _tc_out,
      grid=(nb,),
      in_specs=[
          pl.BlockSpec((NC, _BR, D_H), lambda i: (0, i, 0)),
          pl.BlockSpec((NC, _BR, 128), lambda i: (0, i, 0)),
          pl.BlockSpec((_BR, N_CLS), lambda i: (i, 0)),
          pl.BlockSpec((D_H, N_CLS), lambda i: (0, 0)),
      ],
      out_specs=pl.BlockSpec((_BR, N_CLS), lambda i: (i, 0)),
      out_shape=jax.ShapeDtypeStruct((N, N_CLS), jnp.float32),
  )(s2, deg, r2, W_l2)
  return out
